# Initial kernel scaffold; baseline (speedup 1.0000x reference)
import jax
import jax.numpy as jnp
from jax.experimental import pallas as pl


def kernel(pred, target, batch_idx, num_graphs):
    raise NotImplementedError("write your pallas kernel here")


# speedup vs baseline: 3.1232x; 3.1232x over previous
import jax
import jax.numpy as jnp
from jax.experimental import pallas as pl
from jax.experimental.pallas import tpu as pltpu

_BLOCK_ROWS = 2000


def _sse_block_kernel(p_ref, t_ref, o_ref):
    @pl.when(pl.program_id(0) == 0)
    def _():
        o_ref[0] = 0.0

    d = p_ref[...] - t_ref[...]
    o_ref[0] += jnp.sum(d * d)


def kernel(pred, target, batch_idx, num_graphs):
    del batch_idx
    n_rows, n_feat = pred.shape
    grid = (n_rows // _BLOCK_ROWS,)
    total = pl.pallas_call(
        _sse_block_kernel,
        grid=grid,
        in_specs=[
            pl.BlockSpec((_BLOCK_ROWS, n_feat), lambda i: (i, 0)),
            pl.BlockSpec((_BLOCK_ROWS, n_feat), lambda i: (i, 0)),
        ],
        out_specs=pl.BlockSpec(
            (1,), lambda i: (0,), memory_space=pltpu.SMEM
        ),
        out_shape=jax.ShapeDtypeStruct((1,), jnp.float32),
    )(pred, target)
    return total[0] / num_graphs


# speedup vs baseline: 4.8169x; 1.5423x over previous
import jax
import jax.numpy as jnp
from jax.experimental import pallas as pl
from jax.experimental.pallas import tpu as pltpu

_BLOCK_ROWS = 10000


def _sse_block_kernel(p_ref, t_ref, o_ref):
    @pl.when(pl.program_id(0) == 0)
    def _():
        o_ref[0] = 0.0

    d = p_ref[...] - t_ref[...]
    o_ref[0] += jnp.sum(d * d)


def kernel(pred, target, batch_idx, num_graphs):
    del batch_idx
    n_rows, n_feat = pred.shape
    grid = (n_rows // _BLOCK_ROWS,)
    total = pl.pallas_call(
        _sse_block_kernel,
        grid=grid,
        in_specs=[
            pl.BlockSpec((_BLOCK_ROWS, n_feat), lambda i: (i, 0)),
            pl.BlockSpec((_BLOCK_ROWS, n_feat), lambda i: (i, 0)),
        ],
        out_specs=pl.BlockSpec(
            (1,), lambda i: (0,), memory_space=pltpu.SMEM
        ),
        out_shape=jax.ShapeDtypeStruct((1,), jnp.float32),
    )(pred, target)
    return total[0] / num_graphs


# speedup vs baseline: 4.8437x; 1.0056x over previous
import jax
import jax.numpy as jnp
from jax.experimental import pallas as pl
from jax.experimental.pallas import tpu as pltpu

_BLOCK_ROWS = 20000


def _sse_block_kernel(p_ref, t_ref, o_ref):
    @pl.when(pl.program_id(0) == 0)
    def _():
        o_ref[0] = 0.0

    d = p_ref[...] - t_ref[...]
    o_ref[0] += jnp.sum(d * d)


def kernel(pred, target, batch_idx, num_graphs):
    del batch_idx
    n_rows, n_feat = pred.shape
    grid = (n_rows // _BLOCK_ROWS,)
    total = pl.pallas_call(
        _sse_block_kernel,
        grid=grid,
        in_specs=[
            pl.BlockSpec((_BLOCK_ROWS, n_feat), lambda i: (i, 0)),
            pl.BlockSpec((_BLOCK_ROWS, n_feat), lambda i: (i, 0)),
        ],
        out_specs=pl.BlockSpec(
            (1,), lambda i: (0,), memory_space=pltpu.SMEM
        ),
        out_shape=jax.ShapeDtypeStruct((1,), jnp.float32),
    )(pred, target)
    return total[0] / num_graphs


# speedup vs baseline: 5.2008x; 1.0737x over previous
import jax
import jax.numpy as jnp
from jax.experimental import pallas as pl
from jax.experimental.pallas import tpu as pltpu

_BLOCK_ROWS = 10000


def _sse_block_kernel(ng_ref, p_ref, t_ref, o_ref):
    i = pl.program_id(0)

    @pl.when(i == 0)
    def _():
        o_ref[0] = 0.0

    d = p_ref[...] - t_ref[...]
    o_ref[0] += jnp.sum(d * d)

    @pl.when(i == pl.num_programs(0) - 1)
    def _():
        o_ref[0] = o_ref[0] / ng_ref[0]


def kernel(pred, target, batch_idx, num_graphs):
    del batch_idx
    n_rows, n_feat = pred.shape
    ng = jnp.asarray(num_graphs, jnp.float32).reshape(1)
    grid = (n_rows // _BLOCK_ROWS,)
    total = pl.pallas_call(
        _sse_block_kernel,
        grid=grid,
        in_specs=[
            pl.BlockSpec(memory_space=pltpu.SMEM),
            pl.BlockSpec((_BLOCK_ROWS, n_feat), lambda i: (i, 0)),
            pl.BlockSpec((_BLOCK_ROWS, n_feat), lambda i: (i, 0)),
        ],
        out_specs=pl.BlockSpec(
            (1,), lambda i: (0,), memory_space=pltpu.SMEM
        ),
        out_shape=jax.ShapeDtypeStruct((1,), jnp.float32),
    )(ng, pred, target)
    return total[0]
